# Initial kernel scaffold; baseline (speedup 1.0000x reference)
import jax
import jax.numpy as jnp
from jax import lax
from jax.experimental import pallas as pl
from jax.experimental.pallas import tpu as pltpu
from jax.experimental.pallas import tpu_sc as plsc

f32 = jnp.float32
i32 = jnp.int32

NC = 2
NS = 16
NW = NC * NS
K = 80
LANES = 16

N_PAD = 10240


def _deg_body(row_hbm, ew_hbm, degp_hbm, idx_v, ew_v, upd_v, zb_v, acc):
    n_edges = row_hbm.shape[0]
    epw = n_edges // NW
    nwin = epw // K
    c = lax.axis_index("c")
    s = lax.axis_index("s")
    w = c * NS + s
    rows_per_w = N_PAD // NS
    base_r = s * rows_per_w

    zrow = jnp.zeros((1, LANES), f32)
    zr = zb_v.shape[0]

    @pl.loop(0, zr)
    def _(i):
        zb_v[pl.ds(i, 1), :] = zrow

    @pl.loop(0, rows_per_w // zr)
    def _(t):
        pltpu.sync_copy(zb_v, acc.at[pl.ds(base_r + t * zr, zr)])

    plsc.subcore_barrier()

    ebase = w * epw

    @pl.loop(0, nwin)
    def _(t):
        b = ebase + t * K
        pltpu.sync_copy(row_hbm.at[pl.ds(b, K)], idx_v)
        pltpu.sync_copy(ew_hbm.at[pl.ds(b, K)], ew_v)

        @pl.loop(0, K)
        def _(j):
            upd_v[pl.ds(j, 1), :] = jnp.full((1, LANES), ew_v[j], f32)

        pltpu.sync_copy(upd_v, acc.at[idx_v], add=True)

    plsc.subcore_barrier()
    pltpu.sync_copy(acc.at[pl.ds(base_r, rows_per_w)],
                    degp_hbm.at[c, pl.ds(base_r, rows_per_w)])


def _deg_partial(row, ew):
    kern = pl.kernel(
        _deg_body,
        out_type=jax.ShapeDtypeStruct((NC, N_PAD, LANES), f32),
        mesh=plsc.VectorSubcoreMesh(core_axis_name="c", subcore_axis_name="s"),
        scratch_types=[
            pltpu.VMEM((K,), i32),
            pltpu.VMEM((K,), f32),
            pltpu.VMEM((K, LANES), f32),
            pltpu.VMEM((128, LANES), f32),
            pltpu.VMEM_SHARED((N_PAD, LANES), f32),
        ],
    )
    return kern(row, ew)


def _mp_body(xs_hbm, row_hbm, col_hbm, ew_hbm, accp_hbm,
             col_v, row_v, ew_v, g_v, zb_v, acc):
    n_edges = row_hbm.shape[0]
    epw = n_edges // NW
    nwin = epw // K
    c = lax.axis_index("c")
    s = lax.axis_index("s")
    w = c * NS + s
    rows_per_w = N_PAD // NS
    base_r = s * rows_per_w

    @pl.when(c == 0)
    def _():
        pltpu.sync_copy(xs_hbm.at[pl.ds(base_r, rows_per_w)],
                        acc.at[pl.ds(base_r, rows_per_w)])

    @pl.when(c != 0)
    def _():
        zr = zb_v.shape[0]
        zrow = jnp.zeros((1, LANES), f32)

        @pl.loop(0, zr)
        def _(i):
            for cc in range(8):
                zb_v[pl.ds(i, 1), pl.ds(cc * LANES, LANES)] = zrow

        @pl.loop(0, rows_per_w // zr)
        def _(t):
            pltpu.sync_copy(zb_v, acc.at[pl.ds(base_r + t * zr, zr)])

    plsc.subcore_barrier()

    ebase = w * epw

    @pl.loop(0, nwin)
    def _(t):
        b = ebase + t * K
        pltpu.sync_copy(col_hbm.at[pl.ds(b, K)], col_v)
        pltpu.sync_copy(row_hbm.at[pl.ds(b, K)], row_v)
        pltpu.sync_copy(ew_hbm.at[pl.ds(b, K)], ew_v)
        pltpu.sync_copy(xs_hbm.at[col_v], g_v)

        @pl.loop(0, K)
        def _(j):
            bc = jnp.full((1, LANES), ew_v[j], f32)
            for cc in range(8):
                sl = (pl.ds(j, 1), pl.ds(cc * LANES, LANES))
                g_v[sl] = g_v[sl] * bc

        pltpu.sync_copy(g_v, acc.at[row_v], add=True)

    plsc.subcore_barrier()
    pltpu.sync_copy(acc.at[pl.ds(base_r, rows_per_w)],
                    accp_hbm.at[c, pl.ds(base_r, rows_per_w)])


def _mp_partial(xs, row, col, ew):
    kern = pl.kernel(
        _mp_body,
        out_type=jax.ShapeDtypeStruct((NC, N_PAD, 128), f32),
        mesh=plsc.VectorSubcoreMesh(core_axis_name="c", subcore_axis_name="s"),
        scratch_types=[
            pltpu.VMEM((K,), i32),
            pltpu.VMEM((K,), i32),
            pltpu.VMEM((K,), f32),
            pltpu.VMEM((K, 128), f32),
            pltpu.VMEM((128, 128), f32),
            pltpu.VMEM_SHARED((N_PAD, 128), f32),
        ],
    )
    return kern(xs, row, col, ew)


def _mm_body(x_ref, w_ref, o_ref):
    o_ref[...] = jnp.dot(x_ref[...], w_ref[...],
                         preferred_element_type=f32,
                         precision=lax.Precision.HIGHEST)


def _matmul(x_pad, W):
    blk = 1024
    return pl.pallas_call(
        _mm_body,
        grid=(N_PAD // blk,),
        in_specs=[
            pl.BlockSpec((blk, 128), lambda i: (i, 0)),
            pl.BlockSpec((128, 128), lambda i: (0, 0)),
        ],
        out_specs=pl.BlockSpec((blk, 128), lambda i: (i, 0)),
        out_shape=jax.ShapeDtypeStruct((N_PAD, 128), f32),
    )(x_pad, W)


def _scale_body(degp_ref, xw_ref, xs_ref, dinvb_ref):
    deg = 1.0 + degp_ref[0, :, 0:1] + degp_ref[1, :, 0:1]
    dinv = lax.rsqrt(deg)
    xs_ref[...] = dinv * xw_ref[...]
    dinvb_ref[...] = jnp.broadcast_to(dinv, xw_ref.shape)


def _scale(degp, xw):
    blk = 1024
    return pl.pallas_call(
        _scale_body,
        grid=(N_PAD // blk,),
        in_specs=[
            pl.BlockSpec((NC, blk, LANES), lambda i: (0, i, 0)),
            pl.BlockSpec((blk, 128), lambda i: (i, 0)),
        ],
        out_specs=[
            pl.BlockSpec((blk, 128), lambda i: (i, 0)),
            pl.BlockSpec((blk, 128), lambda i: (i, 0)),
        ],
        out_shape=[
            jax.ShapeDtypeStruct((N_PAD, 128), f32),
            jax.ShapeDtypeStruct((N_PAD, 128), f32),
        ],
    )(degp, xw)


def _fin_body(accp_ref, dinvb_ref, b_ref, o_ref):
    o_ref[...] = dinvb_ref[...] * (accp_ref[0] + accp_ref[1]) + b_ref[...]


def _finish(accp, dinvb, b2d):
    blk = 1024
    return pl.pallas_call(
        _fin_body,
        grid=(N_PAD // blk,),
        in_specs=[
            pl.BlockSpec((NC, blk, 128), lambda i: (0, i, 0)),
            pl.BlockSpec((blk, 128), lambda i: (i, 0)),
            pl.BlockSpec((1, 128), lambda i: (0, 0)),
        ],
        out_specs=pl.BlockSpec((blk, 128), lambda i: (i, 0)),
        out_shape=jax.ShapeDtypeStruct((N_PAD, 128), f32),
    )(accp, dinvb, b2d)


def kernel(x, edge_index, edge_weight, W, b):
    n = x.shape[1]
    row = edge_index[0].astype(i32)
    col = edge_index[1].astype(i32)
    ew = edge_weight.astype(f32)

    x_pad = jnp.pad(x[0], ((0, N_PAD - n), (0, 0)))

    degp = _deg_partial(row, ew)
    xw = _matmul(x_pad, W)
    xs, dinvb = _scale(degp, xw)
    accp = _mp_partial(xs, row, col, ew)
    out = _finish(accp, dinvb, b.reshape(1, 128))

    return out[:n].reshape(1, n, -1)


# speedup vs baseline: 12.6920x; 12.6920x over previous
import jax
import jax.numpy as jnp
from jax import lax
from jax.experimental import pallas as pl
from jax.experimental.pallas import tpu as pltpu
from jax.experimental.pallas import tpu_sc as plsc

f32 = jnp.float32
i32 = jnp.int32

NC = 2
NS = 16
NW = NC * NS
K = 80
LANES = 16

N_PAD = 10240


def _deg_body(row_hbm, ew_hbm, degp_hbm, idx_v, ew_v, upd_v, zb_v, acc):
    n_edges = row_hbm.shape[0]
    epw = n_edges // NW
    nwin = epw // K
    c = lax.axis_index("c")
    s = lax.axis_index("s")
    w = c * NS + s
    rows_per_w = N_PAD // NS
    base_r = s * rows_per_w

    zrow = jnp.zeros((1, LANES), f32)
    zr = zb_v.shape[0]

    @pl.loop(0, zr)
    def _(i):
        zb_v[pl.ds(i, 1), :] = zrow

    @pl.loop(0, rows_per_w // zr)
    def _(t):
        pltpu.sync_copy(zb_v, acc.at[pl.ds(base_r + t * zr, zr)])

    plsc.subcore_barrier()

    ebase = w * epw

    @pl.loop(0, nwin)
    def _(t):
        b = ebase + t * K
        pltpu.sync_copy(row_hbm.at[pl.ds(b, K)], idx_v)
        pltpu.sync_copy(ew_hbm.at[pl.ds(b, K)], ew_v)

        @pl.loop(0, K // LANES)
        def _(jj):
            v = ew_v[pl.ds(jj * LANES, LANES)]
            for l in range(LANES):
                upd_v[pl.ds(jj * LANES + l, 1), :] = jnp.full(
                    (1, LANES), v[l], f32)

        pltpu.sync_copy(upd_v, acc.at[idx_v], add=True)

    plsc.subcore_barrier()
    pltpu.sync_copy(acc.at[pl.ds(base_r, rows_per_w)],
                    degp_hbm.at[c, pl.ds(base_r, rows_per_w)])


def _deg_partial(row, ew):
    kern = pl.kernel(
        _deg_body,
        out_type=jax.ShapeDtypeStruct((NC, N_PAD, LANES), f32),
        mesh=plsc.VectorSubcoreMesh(core_axis_name="c", subcore_axis_name="s"),
        scratch_types=[
            pltpu.VMEM((K,), i32),
            pltpu.VMEM((K,), f32),
            pltpu.VMEM((K, LANES), f32),
            pltpu.VMEM((128, LANES), f32),
            pltpu.VMEM_SHARED((N_PAD, LANES), f32),
        ],
    )
    return kern(row, ew)


def _mp_body(xs_hbm, row_hbm, col_hbm, ew_hbm, accp_hbm,
             col_v, row_v, ew_v, g_v, zb_v, acc):
    n_edges = row_hbm.shape[0]
    epw = n_edges // NW
    nwin = epw // K
    c = lax.axis_index("c")
    s = lax.axis_index("s")
    w = c * NS + s
    rows_per_w = N_PAD // NS
    base_r = s * rows_per_w

    @pl.when(c == 0)
    def _():
        pltpu.sync_copy(xs_hbm.at[pl.ds(base_r, rows_per_w)],
                        acc.at[pl.ds(base_r, rows_per_w)])

    @pl.when(c != 0)
    def _():
        zr = zb_v.shape[0]
        zrow = jnp.zeros((1, LANES), f32)

        @pl.loop(0, zr)
        def _(i):
            for cc in range(8):
                zb_v[pl.ds(i, 1), pl.ds(cc * LANES, LANES)] = zrow

        @pl.loop(0, rows_per_w // zr)
        def _(t):
            pltpu.sync_copy(zb_v, acc.at[pl.ds(base_r + t * zr, zr)])

    plsc.subcore_barrier()

    ebase = w * epw

    @pl.loop(0, nwin)
    def _(t):
        b = ebase + t * K
        pltpu.sync_copy(col_hbm.at[pl.ds(b, K)], col_v)
        pltpu.sync_copy(row_hbm.at[pl.ds(b, K)], row_v)
        pltpu.sync_copy(ew_hbm.at[pl.ds(b, K)], ew_v)
        pltpu.sync_copy(xs_hbm.at[col_v], g_v)

        @pl.loop(0, K // LANES)
        def _(jj):
            v = ew_v[pl.ds(jj * LANES, LANES)]
            for l in range(LANES):
                bc = jnp.full((1, LANES), v[l], f32)
                j = jj * LANES + l
                for cc in range(8):
                    sl = (pl.ds(j, 1), pl.ds(cc * LANES, LANES))
                    g_v[sl] = g_v[sl] * bc

        pltpu.sync_copy(g_v, acc.at[row_v], add=True)

    plsc.subcore_barrier()
    pltpu.sync_copy(acc.at[pl.ds(base_r, rows_per_w)],
                    accp_hbm.at[c, pl.ds(base_r, rows_per_w)])


def _mp_partial(xs, row, col, ew):
    kern = pl.kernel(
        _mp_body,
        out_type=jax.ShapeDtypeStruct((NC, N_PAD, 128), f32),
        mesh=plsc.VectorSubcoreMesh(core_axis_name="c", subcore_axis_name="s"),
        scratch_types=[
            pltpu.VMEM((K,), i32),
            pltpu.VMEM((K,), i32),
            pltpu.VMEM((K,), f32),
            pltpu.VMEM((K, 128), f32),
            pltpu.VMEM((128, 128), f32),
            pltpu.VMEM_SHARED((N_PAD, 128), f32),
        ],
    )
    return kern(xs, row, col, ew)


def _mm_body(x_ref, w_ref, o_ref):
    o_ref[...] = jnp.dot(x_ref[...], w_ref[...],
                         preferred_element_type=f32,
                         precision=lax.Precision.HIGHEST)


def _matmul(x_pad, W):
    blk = 1024
    return pl.pallas_call(
        _mm_body,
        grid=(N_PAD // blk,),
        in_specs=[
            pl.BlockSpec((blk, 128), lambda i: (i, 0)),
            pl.BlockSpec((128, 128), lambda i: (0, 0)),
        ],
        out_specs=pl.BlockSpec((blk, 128), lambda i: (i, 0)),
        out_shape=jax.ShapeDtypeStruct((N_PAD, 128), f32),
    )(x_pad, W)


def _scale_body(degp_ref, xw_ref, xs_ref, dinvb_ref):
    deg = 1.0 + degp_ref[0, :, 0:1] + degp_ref[1, :, 0:1]
    dinv = lax.rsqrt(deg)
    xs_ref[...] = dinv * xw_ref[...]
    dinvb_ref[...] = jnp.broadcast_to(dinv, xw_ref.shape)


def _scale(degp, xw):
    blk = 1024
    return pl.pallas_call(
        _scale_body,
        grid=(N_PAD // blk,),
        in_specs=[
            pl.BlockSpec((NC, blk, LANES), lambda i: (0, i, 0)),
            pl.BlockSpec((blk, 128), lambda i: (i, 0)),
        ],
        out_specs=[
            pl.BlockSpec((blk, 128), lambda i: (i, 0)),
            pl.BlockSpec((blk, 128), lambda i: (i, 0)),
        ],
        out_shape=[
            jax.ShapeDtypeStruct((N_PAD, 128), f32),
            jax.ShapeDtypeStruct((N_PAD, 128), f32),
        ],
    )(degp, xw)


def _fin_body(accp_ref, dinvb_ref, b_ref, o_ref):
    o_ref[...] = dinvb_ref[...] * (accp_ref[0] + accp_ref[1]) + b_ref[...]


def _finish(accp, dinvb, b2d):
    blk = 1024
    return pl.pallas_call(
        _fin_body,
        grid=(N_PAD // blk,),
        in_specs=[
            pl.BlockSpec((NC, blk, 128), lambda i: (0, i, 0)),
            pl.BlockSpec((blk, 128), lambda i: (i, 0)),
            pl.BlockSpec((1, 128), lambda i: (0, 0)),
        ],
        out_specs=pl.BlockSpec((blk, 128), lambda i: (i, 0)),
        out_shape=jax.ShapeDtypeStruct((N_PAD, 128), f32),
    )(accp, dinvb, b2d)


def kernel(x, edge_index, edge_weight, W, b):
    n = x.shape[1]
    row = edge_index[0].astype(i32)
    col = edge_index[1].astype(i32)
    ew = edge_weight.astype(f32)

    x_pad = jnp.pad(x[0], ((0, N_PAD - n), (0, 0)))

    degp = _deg_partial(row, ew)
    xw = _matmul(x_pad, W)
    xs, dinvb = _scale(degp, xw)
    accp = _mp_partial(xs, row, col, ew)
    out = _finish(accp, dinvb, b.reshape(1, 128))

    return out[:n].reshape(1, n, -1)
